# Initial kernel scaffold; baseline (speedup 1.0000x reference)
import jax
import jax.numpy as jnp
from jax.experimental import pallas as pl


def kernel(x, emb_weight, emb_bias):
    raise NotImplementedError("write your pallas kernel here")


# speedup vs baseline: 1.8998x; 1.8998x over previous
import functools

import jax
import jax.numpy as jnp
from jax import lax
from jax.experimental import pallas as pl
from jax.experimental.pallas import tpu as pltpu
from jax.experimental.pallas import tpu_sc as plsc

_B = 16384
_F = 26
_D = 32
_XPAD = 128

_INFO = plsc.get_sparse_core_info()
_NC = _INFO.num_cores
_NS = _INFO.num_subcores
_NW = _NC * _NS
_BPW = _B // _NW
_NB = 64
_NCHUNK = _BPW // _NB
_NIDX = _NB * _F + 6


def _sc_body(x_hbm, tab_hbm, bias_hbm, out_hbm, xrow_v, idx_v, rows_v, out_v, bias_v, sem):
    wid = lax.axis_index("s") * _NC + lax.axis_index("c")
    pltpu.sync_copy(bias_hbm, bias_v)

    def chunk_body(c, _):
        base = wid * _BPW + c * _NB
        pltpu.sync_copy(x_hbm.at[pl.ds(base * _XPAD, _NB * _XPAD)], xrow_v)

        def pack_body(b, _):
            v1 = xrow_v[pl.ds(b * _XPAD + 16, 16)]
            idx_v[pl.ds(b * _F + 16, 16)] = v1
            v0 = xrow_v[pl.ds(b * _XPAD, 16)]
            idx_v[pl.ds(b * _F, 16)] = v0
            return 0

        lax.fori_loop(0, _NB, pack_body, 0)
        pltpu.async_copy(tab_hbm.at[idx_v], rows_v, sem).wait()

        def row_body(b, _):
            rb = b * _F
            a0 = bias_v[pl.ds(0, 16)]
            a1 = bias_v[pl.ds(16, 16)]
            for f in range(_F):
                a0 = a0 + rows_v[rb + f, pl.ds(0, 16)]
                a1 = a1 + rows_v[rb + f, pl.ds(16, 16)]
            out_v[b, pl.ds(0, 16)] = a0
            out_v[b, pl.ds(16, 16)] = a1
            return 0

        lax.fori_loop(0, _NB, row_body, 0)
        pltpu.sync_copy(out_v, out_hbm.at[pl.ds(base, _NB), :])
        return 0

    lax.fori_loop(0, _NCHUNK, chunk_body, 0)


def kernel(x, emb_weight, emb_bias):
    x_pad = jnp.pad(x.astype(jnp.int32), ((0, 0), (0, _XPAD - _F))).reshape(-1)
    mesh = plsc.VectorSubcoreMesh(core_axis_name="c", subcore_axis_name="s")
    k = functools.partial(
        pl.kernel,
        mesh=mesh,
        out_type=jax.ShapeDtypeStruct((_B, _XPAD), jnp.float32),
        scratch_types=[
            pltpu.VMEM((_NB * _XPAD,), jnp.int32),
            pltpu.VMEM((_NIDX,), jnp.int32),
            pltpu.VMEM((_NIDX, _D), jnp.float32),
            pltpu.VMEM((_NB, _XPAD), jnp.float32),
            pltpu.VMEM((_D,), jnp.float32),
            pltpu.SemaphoreType.DMA,
        ],
        compiler_params=pltpu.CompilerParams(use_tc_tiling_on_sc=False),
    )(_sc_body)
    out_pad = k(x_pad, emb_weight, emb_bias)
    return out_pad[:, :_D]


# speedup vs baseline: 1.9022x; 1.0013x over previous
import functools

import jax
import jax.numpy as jnp
from jax import lax
from jax.experimental import pallas as pl
from jax.experimental.pallas import tpu as pltpu
from jax.experimental.pallas import tpu_sc as plsc

_B = 16384
_F = 26
_D = 32
_XPAD = 128

_INFO = plsc.get_sparse_core_info()
_NC = _INFO.num_cores
_NS = _INFO.num_subcores
_NW = _NC * _NS
_BPW = _B // _NW
_NB = 64
_NCHUNK = _BPW // _NB
_NIDX = _NB * _F + 6


def _sc_body(x_hbm, tab_hbm, bias_hbm, out_hbm, xrow_v, idx_v, rows_v, out_v, bias_v, sem):
    wid = lax.axis_index("s") * _NC + lax.axis_index("c")
    pltpu.sync_copy(bias_hbm, bias_v)

    def chunk_body(c, _):
        base = wid * _BPW + c * _NB
        pltpu.sync_copy(x_hbm.at[pl.ds(base, _NB), :], xrow_v)

        def pack_body(b, _):
            v1 = xrow_v[b, pl.ds(16, 16)]
            idx_v[pl.ds(b * _F + 16, 16)] = v1
            v0 = xrow_v[b, pl.ds(0, 16)]
            idx_v[pl.ds(b * _F, 16)] = v0
            return 0

        lax.fori_loop(0, _NB, pack_body, 0)
        pltpu.async_copy(tab_hbm.at[idx_v], rows_v, sem).wait()

        def row_body(b, _):
            rb = b * _F
            a0 = bias_v[pl.ds(0, 16)]
            a1 = bias_v[pl.ds(16, 16)]
            for f in range(_F):
                a0 = a0 + rows_v[rb + f, pl.ds(0, 16)]
                a1 = a1 + rows_v[rb + f, pl.ds(16, 16)]
            out_v[b, pl.ds(0, 16)] = a0
            out_v[b, pl.ds(16, 16)] = a1
            return 0

        lax.fori_loop(0, _NB, row_body, 0)
        pltpu.sync_copy(out_v, out_hbm.at[pl.ds(base, _NB), :])
        return 0

    lax.fori_loop(0, _NCHUNK, chunk_body, 0)


def kernel(x, emb_weight, emb_bias):
    x_pad = jnp.pad(x.astype(jnp.int32), ((0, 0), (0, _XPAD - _F)))
    mesh = plsc.VectorSubcoreMesh(core_axis_name="c", subcore_axis_name="s")
    k = functools.partial(
        pl.kernel,
        mesh=mesh,
        out_type=jax.ShapeDtypeStruct((_B, _XPAD), jnp.float32),
        scratch_types=[
            pltpu.VMEM((_NB, _XPAD), jnp.int32),
            pltpu.VMEM((_NIDX,), jnp.int32),
            pltpu.VMEM((_NIDX, _D), jnp.float32),
            pltpu.VMEM((_NB, _XPAD), jnp.float32),
            pltpu.VMEM((_D,), jnp.float32),
            pltpu.SemaphoreType.DMA,
        ],
        compiler_params=pltpu.CompilerParams(use_tc_tiling_on_sc=False),
    )(_sc_body)
    out_pad = k(x_pad, emb_weight, emb_bias)
    return out_pad[:, :_D]
